# baseline (speedup 1.0000x reference)
import functools

import jax
import jax.numpy as jnp
from jax import lax
from jax.experimental import pallas as pl
from jax.experimental.pallas import tpu as pltpu
from jax.experimental.pallas import tpu_sc as plsc

_BATCH = 4
_NC = 2
_NS = 16
_NW = _NC * _NS

_SPLIT = 5120
_BLK = 1024
_CHUNK = 32
_NBUF = 2


def _tc_body(enc_ref, out_ref):
    blk = enc_ref[...]
    out_ref[...] = jnp.broadcast_to(blk[None], (_BATCH,) + blk.shape)


def _tc_part(encoding):
    rows, dim = encoding.shape
    return pl.pallas_call(
        _tc_body,
        grid=(rows // _BLK,),
        in_specs=[pl.BlockSpec((_BLK, dim), lambda i: (i, 0))],
        out_specs=pl.BlockSpec((_BATCH, _BLK, dim), lambda i: (0, i, 0)),
        out_shape=jax.ShapeDtypeStruct((_BATCH, rows, dim), encoding.dtype),
    )(encoding)


def _sc_part(encoding):
    rows, dim = encoding.shape
    rows_per_w = rows // _NW
    n_chunks = rows_per_w // _CHUNK

    mesh = plsc.VectorSubcoreMesh(core_axis_name="c", subcore_axis_name="s")

    @functools.partial(
        pl.kernel,
        mesh=mesh,
        out_type=jax.ShapeDtypeStruct((_BATCH, rows, dim), encoding.dtype),
        scratch_types=[
            pltpu.VMEM((_NBUF, _CHUNK, dim), jnp.float32),
            pltpu.SemaphoreType.DMA,
            pltpu.SemaphoreType.DMA,
        ],
    )
    def sc_copy(enc_hbm, out_hbm, bufs, in_sem, out_sem):
        wid = lax.axis_index("s") * _NC + lax.axis_index("c")
        base = wid * rows_per_w

        def in_copy(i, slot):
            return pltpu.make_async_copy(
                enc_hbm.at[pl.ds(base + i * _CHUNK, _CHUNK)],
                bufs.at[slot],
                in_sem,
            )

        def out_copies(i, slot):
            return [
                pltpu.make_async_copy(
                    bufs.at[slot],
                    out_hbm.at[b, pl.ds(base + i * _CHUNK, _CHUNK)],
                    out_sem,
                )
                for b in range(_BATCH)
            ]

        in_copy(0, 0).start()

        def step(i, _):
            slot = lax.rem(i, _NBUF)
            in_copy(i, slot).wait()

            @pl.when(i + 1 < n_chunks)
            def _():
                in_copy(i + 1, lax.rem(i + 1, _NBUF)).start()

            cs = out_copies(i, slot)
            for c in cs:
                c.start()
            for c in cs:
                c.wait()
            return 0

        lax.fori_loop(0, n_chunks, step, 0)

    return sc_copy(encoding)


def kernel(encoding, batch_size, seq_len):
    tc_out = _tc_part(encoding[:_SPLIT])
    sc_out = _sc_part(encoding[_SPLIT:])
    return jnp.concatenate([tc_out, sc_out], axis=1)


# speedup vs baseline: 2.2379x; 2.2379x over previous
import functools

import jax
import jax.numpy as jnp
from jax import lax
from jax.experimental import pallas as pl
from jax.experimental.pallas import tpu as pltpu
from jax.experimental.pallas import tpu_sc as plsc

_BATCH = 4
_NC = 2
_NS = 16
_NW = _NC * _NS

_CHUNK = 32
_NBUF = 2


def kernel(encoding, batch_size, seq_len):
    max_len, dim = encoding.shape
    rows_per_w = max_len // _NW
    n_chunks = rows_per_w // _CHUNK

    mesh = plsc.VectorSubcoreMesh(core_axis_name="c", subcore_axis_name="s")

    @functools.partial(
        pl.kernel,
        mesh=mesh,
        out_type=jax.ShapeDtypeStruct((_BATCH, max_len, dim), encoding.dtype),
        scratch_types=[
            pltpu.VMEM((_NBUF, _CHUNK, dim), jnp.float32),
            pltpu.SemaphoreType.DMA,
        ]
        + [pltpu.SemaphoreType.DMA for _ in range(_NBUF)],
    )
    def sc_copy(enc_hbm, out_hbm, bufs, in_sem, *out_sems):
        wid = lax.axis_index("s") * _NC + lax.axis_index("c")
        base = wid * rows_per_w

        def in_copy(i, slot):
            return pltpu.make_async_copy(
                enc_hbm.at[pl.ds(base + i * _CHUNK, _CHUNK)],
                bufs.at[slot],
                in_sem,
            )

        def out_copies(i, slot):
            return [
                pltpu.make_async_copy(
                    bufs.at[slot],
                    out_hbm.at[b, pl.ds(base + i * _CHUNK, _CHUNK)],
                    out_sems[slot],
                )
                for b in range(_BATCH)
            ]

        in_copy(0, 0).start()
        for i in range(n_chunks):
            slot = i % _NBUF
            in_copy(i, slot).wait()
            for c in out_copies(i, slot):
                c.start()
            j = i + 1
            if j < n_chunks:
                jslot = j % _NBUF
                if j >= _NBUF:
                    for c in out_copies(j - _NBUF, jslot):
                        c.wait()
                in_copy(j, jslot).start()
        for i in range(max(0, n_chunks - _NBUF), n_chunks):
            for c in out_copies(i, i % _NBUF):
                c.wait()

    return sc_copy(encoding)


# speedup vs baseline: 2.3264x; 1.0396x over previous
import functools

import jax
import jax.numpy as jnp
from jax import lax
from jax.experimental import pallas as pl
from jax.experimental.pallas import tpu as pltpu
from jax.experimental.pallas import tpu_sc as plsc

_BATCH = 4
_NC = 2
_NS = 16
_NW = _NC * _NS


_CHUNK = 32
_NBUF = 3


def kernel(encoding, batch_size, seq_len):
    max_len, dim = encoding.shape
    rows_per_w = max_len // _NW
    n_chunks = rows_per_w // _CHUNK

    mesh = plsc.VectorSubcoreMesh(core_axis_name="c", subcore_axis_name="s")

    @functools.partial(
        pl.kernel,
        mesh=mesh,
        out_type=jax.ShapeDtypeStruct((_BATCH, max_len, dim), encoding.dtype),
        scratch_types=[
            pltpu.VMEM((_NBUF, _CHUNK, dim), jnp.float32),
            pltpu.SemaphoreType.DMA,
            pltpu.SemaphoreType.DMA,
        ],
    )
    def sc_copy(enc_hbm, out_hbm, bufs, in_sem, out_sem):
        wid = lax.axis_index("s") * _NC + lax.axis_index("c")
        base = wid * rows_per_w

        def in_copy(i, slot):
            return pltpu.make_async_copy(
                enc_hbm.at[pl.ds(base + i * _CHUNK, _CHUNK)],
                bufs.at[slot],
                in_sem,
            )

        def out_copies(i, slot):
            return [
                pltpu.make_async_copy(
                    bufs.at[slot],
                    out_hbm.at[b, pl.ds(base + i * _CHUNK, _CHUNK)],
                    out_sem,
                )
                for b in range(_BATCH)
            ]

        in_copy(0, 0).start()

        def step(i, _):
            slot = lax.rem(i, _NBUF)
            in_copy(i, slot).wait()

            @pl.when(i + 1 < n_chunks)
            def _():
                in_copy(i + 1, lax.rem(i + 1, _NBUF)).start()

            cs = out_copies(i, slot)
            for c in cs:
                c.start()
            for c in cs:
                c.wait()
            return 0

        lax.fori_loop(0, n_chunks, step, 0)

    return sc_copy(encoding)


# speedup vs baseline: 2.3589x; 1.0140x over previous
import functools

import jax
import jax.numpy as jnp
from jax import lax
from jax.experimental import pallas as pl
from jax.experimental.pallas import tpu as pltpu
from jax.experimental.pallas import tpu_sc as plsc

_BATCH = 4
_NC = 2
_NS = 16
_NW = _NC * _NS


_CHUNK = 32
_NBUF = 2


def kernel(encoding, batch_size, seq_len):
    max_len, dim = encoding.shape
    rows_per_w = max_len // _NW
    n_chunks = rows_per_w // _CHUNK

    mesh = plsc.VectorSubcoreMesh(core_axis_name="c", subcore_axis_name="s")

    @functools.partial(
        pl.kernel,
        mesh=mesh,
        out_type=jax.ShapeDtypeStruct((_BATCH, max_len, dim), encoding.dtype),
        scratch_types=[
            pltpu.VMEM((_NBUF, _CHUNK, dim), jnp.float32),
            pltpu.SemaphoreType.DMA,
            pltpu.SemaphoreType.DMA,
        ],
    )
    def sc_copy(enc_hbm, out_hbm, bufs, in_sem, out_sem):
        wid = lax.axis_index("s") * _NC + lax.axis_index("c")
        base = wid * rows_per_w

        def in_copy(i, slot):
            return pltpu.make_async_copy(
                enc_hbm.at[pl.ds(base + i * _CHUNK, _CHUNK)],
                bufs.at[slot],
                in_sem,
            )

        def out_copies(i, slot):
            return [
                pltpu.make_async_copy(
                    bufs.at[slot],
                    out_hbm.at[b, pl.ds(base + i * _CHUNK, _CHUNK)],
                    out_sem,
                )
                for b in range(_BATCH)
            ]

        in_copy(0, 0).start()

        def step(i, _):
            slot = lax.rem(i, _NBUF)
            in_copy(i, slot).wait()

            @pl.when(i + 1 < n_chunks)
            def _():
                in_copy(i + 1, lax.rem(i + 1, _NBUF)).start()

            cs = out_copies(i, slot)
            for c in cs:
                c.start()
            for c in cs:
                c.wait()
            return 0

        lax.fori_loop(0, n_chunks, step, 0)

    return sc_copy(encoding)
